# Initial kernel scaffold; baseline (speedup 1.0000x reference)
import jax
import jax.numpy as jnp
from jax.experimental import pallas as pl


def kernel(x, edge_index, batch, W1, as1, ad1, b1, g1, be1, W2, as2, ad2, b2, g2, be2, W3, as3, ad3, b3, g3, be3, fcW, fcb):
    raise NotImplementedError("write your pallas kernel here")


# speedup vs baseline: 1.0011x; 1.0011x over previous
import functools

import jax
import jax.numpy as jnp
from jax.experimental import pallas as pl
from jax.experimental.pallas import tpu as pltpu

N = 10000
E = 320000
F_IN = 128
C = 256
H = 4
G = 128
NHID = 256


def _bn(x, g, b):
    m = jnp.mean(x, axis=0)
    v = jnp.var(x, axis=0)
    return (x - m) / jnp.sqrt(v + 1e-5) * g + b


def _gat(x, ei, W, a_src, a_dst, bias):
    h = (x @ W).reshape(x.shape[0], H, C)
    src, dst = ei[0], ei[1]
    alpha_src = jnp.sum(h * a_src[None], axis=-1)
    alpha_dst = jnp.sum(h * a_dst[None], axis=-1)
    alpha = alpha_src[src] + alpha_dst[dst]
    alpha = jax.nn.leaky_relu(alpha, 0.2)
    amax = jax.ops.segment_max(alpha, dst, num_segments=N)
    amax = jnp.where(jnp.isfinite(amax), amax, 0.0)
    ex = jnp.exp(alpha - amax[dst])
    denom = jax.ops.segment_sum(ex, dst, num_segments=N)
    coef = ex / (denom[dst] + 1e-16)
    msg = h[src] * coef[:, :, None]
    out = jax.ops.segment_sum(msg, dst, num_segments=N)
    return jnp.mean(out, axis=1) + bias


def _pool_fc_body(h_ref, batch_ref, fcW_ref, fcb_ref, out_ref, acc_ref):
    i = pl.program_id(0)
    nb = pl.num_programs(0)

    @pl.when(i == 0)
    def _init():
        acc_ref[...] = jnp.zeros_like(acc_ref)

    hblk = h_ref[...]
    b = batch_ref[...].reshape(1, -1)
    onehot = (b.T == jax.lax.broadcasted_iota(jnp.int32, (1, G), 1)).astype(jnp.float32)
    acc_ref[:G, :] += jnp.dot(onehot.T, hblk, preferred_element_type=jnp.float32)
    acc_ref[G:, :1] += jnp.sum(onehot, axis=0, keepdims=True).T

    @pl.when(i == nb - 1)
    def _fin():
        s = acc_ref[:G, :]
        cnt = acc_ref[G:, :1]
        rep = s / jnp.maximum(cnt, 1.0)
        out_ref[...] = jnp.dot(rep, fcW_ref[...],
                               preferred_element_type=jnp.float32) + fcb_ref[...]


def _pool_fc(h, batch, fcW, fcb):
    BN = 2000
    grid = (N // BN,)
    return pl.pallas_call(
        _pool_fc_body,
        grid=grid,
        in_specs=[
            pl.BlockSpec((BN, C), lambda i: (i, 0)),
            pl.BlockSpec((1, 1, BN), lambda i: (i, 0, 0)),
            pl.BlockSpec((C, NHID), lambda i: (0, 0)),
            pl.BlockSpec((1, NHID), lambda i: (0, 0)),
        ],
        out_specs=pl.BlockSpec((G, NHID), lambda i: (0, 0)),
        out_shape=jax.ShapeDtypeStruct((G, NHID), jnp.float32),
        scratch_shapes=[pltpu.VMEM((2 * G, C), jnp.float32)],
    )(h, batch.reshape(N // BN, 1, BN), fcW, fcb.reshape(1, NHID))


def kernel(x, edge_index, batch, W1, as1, ad1, b1, g1, be1, W2, as2, ad2, b2,
           g2, be2, W3, as3, ad3, b3, g3, be3, fcW, fcb):
    loop = jnp.arange(N, dtype=edge_index.dtype)
    ei = jnp.concatenate([edge_index, jnp.stack([loop, loop])], axis=1)
    h = jax.nn.relu(_bn(_gat(x, ei, W1, as1, ad1, b1), g1, be1))
    h = jax.nn.relu(_bn(_gat(h, ei, W2, as2, ad2, b2), g2, be2))
    h = jax.nn.relu(_bn(_gat(h, ei, W3, as3, ad3, b3), g3, be3))
    return _pool_fc(h, batch, fcW, fcb)


# speedup vs baseline: 17.7849x; 17.7657x over previous
import functools

import jax
import jax.numpy as jnp
from jax import lax
from jax.experimental import pallas as pl
from jax.experimental.pallas import tpu as pltpu
from jax.experimental.pallas import tpu_sc as plsc

N = 10000
E = 320000
ET = E + N
F_IN = 128
C = 256
H = 4
G = 128
NHID = 256

NW = 32
NPW = 320
NPAD = NW * NPW
RPW = 384
ABUF = 128
SSW = ABUF + 8
SSLEN = ((ET + SSW + 7) // 8) * 8
EPS_BN = 1e-5 * (H * H)


def _sc_body(h_hbm, asrc_hbm, adst_hbm, ss_hbm, rp_hbm, out_hbm,
             asrc_v, adst_v, rp_v, ssw_v, albuf, rows_v, accv, rowv, sem):
    wid = lax.axis_index("s") * 2 + lax.axis_index("c")
    lanes = lax.iota(jnp.int32, 16)
    zero16 = jnp.zeros((16,), jnp.float32)

    pltpu.sync_copy(asrc_hbm, asrc_v)
    pltpu.sync_copy(adst_hbm.at[pl.ds(pl.multiple_of(wid * (NPW * 4), 8),
                                      NPW * 4)],
                    adst_v.at[pl.ds(0, NPW * 4)])
    pltpu.sync_copy(rp_hbm.at[wid], rp_v)

    def node_body(nl, _):
        rpv = rp_v[pl.ds(nl, 16)]
        p0 = rpv[0]
        p1 = rpv[1]
        deg = p1 - p0
        ad4 = adst_v[pl.ds(nl * 4, 16)]
        ad = [ad4[hh] for hh in range(H)]

        for hh in range(H):
            for j in range(16):
                accv[hh, pl.ds(j * 16, 16)] = zero16

        def wbody(w, carry):
            m = carry[0:H]
            d = carry[H:2 * H]
            e0 = p0 + w * ABUF
            wlen = jnp.minimum(p1 - e0, ABUF)
            a0 = pl.multiple_of((e0 // 8) * 8, 8)
            skew = e0 - a0
            pltpu.sync_copy(ss_hbm.at[pl.ds(a0, SSW)], ssw_v.at[pl.ds(0, SSW)])
            nch = (wlen + 15) // 16

            def p1_chunk(cc, mx):
                pos = skew + cc * 16 + lanes
                idxv = plsc.load_gather(ssw_v, [pos])
                valid = (cc * 16 + lanes) < wlen
                base4 = idxv * 4
                res = []
                for hh in range(H):
                    av = plsc.load_gather(asrc_v, [base4 + hh])
                    a = av + ad[hh]
                    a = jnp.maximum(a, 0.2 * a)
                    a = jnp.where(valid, a, -1e30)
                    albuf[hh, pl.ds(cc * 16, 16)] = a
                    res.append(jnp.maximum(mx[hh], jnp.max(a)))
                return tuple(res)

            mn = lax.fori_loop(0, nch, p1_chunk, tuple(m))

            rv = [jnp.exp(m[hh] - mn[hh] + zero16) for hh in range(H)]
            d = tuple(d[hh] * rv[hh] for hh in range(H))

            @pl.when(w > 0)
            def _rescale_acc():
                for hh in range(H):
                    for j in range(16):
                        accv[hh, pl.ds(j * 16, 16)] = (
                            accv[hh, pl.ds(j * 16, 16)] * rv[hh])

            def p2_chunk(cc, dn):
                res = []
                for hh in range(H):
                    a = albuf[hh, pl.ds(cc * 16, 16)]
                    ex = jnp.exp(a - mn[hh])
                    albuf[hh, pl.ds(cc * 16, 16)] = ex
                    res.append(dn[hh] + ex)
                return tuple(res)

            d = lax.fori_loop(0, nch, p2_chunk, d)

            def p3_chunk(cc, _c):
                pos = skew + cc * 16 + lanes
                idxv = plsc.load_gather(ssw_v, [pos])
                pltpu.async_copy(h_hbm.at[idxv], rows_v, sem).wait()
                for jh in range(2):
                    accs = [accv[hh, pl.ds((jh * 8 + j) * 16, 16)]
                            for hh in range(H) for j in range(8)]

                    def ebody(e, accs):
                        accs = list(accs)
                        k = 0
                        for hh in range(H):
                            exs = albuf[hh, pl.ds(cc * 16 + e, 16)][0]
                            for j in range(8):
                                off = hh * 256 + (jh * 8 + j) * 16
                                accs[k] = (accs[k]
                                           + exs * rows_v[e, pl.ds(off, 16)])
                                k += 1
                        return tuple(accs)

                    accs = lax.fori_loop(0, 16, ebody, tuple(accs))
                    k = 0
                    for hh in range(H):
                        for j in range(8):
                            accv[hh, pl.ds((jh * 8 + j) * 16, 16)] = accs[k]
                            k += 1
                return 0

            lax.fori_loop(0, nch, p3_chunk, 0)
            return tuple(mn) + tuple(d)

        init = tuple(jnp.float32(-1e30) for _ in range(H)) + tuple(
            zero16 for _ in range(H))
        carry = lax.fori_loop(0, (deg + ABUF - 1) // ABUF, wbody, init)
        inv = [1.0 / (jnp.sum(carry[H + hh]) + 1e-16 + zero16)
               for hh in range(H)]

        for j in range(16):
            acc = accv[0, pl.ds(j * 16, 16)] * inv[0]
            for hh in range(1, H):
                acc = acc + accv[hh, pl.ds(j * 16, 16)] * inv[hh]
            rowv[pl.ds(j * 16, 16)] = acc
        pltpu.sync_copy(rowv, out_hbm.at[wid * NPW + nl])
        return 0

    lax.fori_loop(0, NPW, node_body, 0)


def _sc_layer(h, asrc_flat, adst_flat, ss_pad, rp):
    mesh = plsc.VectorSubcoreMesh(core_axis_name="c", subcore_axis_name="s",
                                  num_cores=2, num_subcores=16)
    f = functools.partial(
        pl.kernel,
        out_type=jax.ShapeDtypeStruct((NPAD, C), jnp.float32),
        mesh=mesh,
        scratch_types=[
            pltpu.VMEM((NPAD * 4,), jnp.float32),
            pltpu.VMEM((NPW * 4 + 128,), jnp.float32),
            pltpu.VMEM((RPW,), jnp.int32),
            pltpu.VMEM((256,), jnp.int32),
            pltpu.VMEM((H, ABUF + 16), jnp.float32),
            pltpu.VMEM((16, H * C), jnp.float32),
            pltpu.VMEM((H, C), jnp.float32),
            pltpu.VMEM((C,), jnp.float32),
            pltpu.SemaphoreType.DMA,
        ],
        compiler_params=pltpu.CompilerParams(needs_layout_passes=False),
    )(_sc_body)
    return f(h, asrc_flat, adst_flat, ss_pad, rp)


def _mm_body(x_ref, W_ref, as_ref, ad_ref, h_ref, asrc_ref, adst_ref):
    xb = x_ref[...]
    hb = jnp.dot(xb, W_ref[...], preferred_element_type=jnp.float32)
    h_ref[...] = hb
    srcs, dsts = [], []
    for hh in range(H):
        hs = hb[:, hh * C:(hh + 1) * C]
        srcs.append(jnp.sum(hs * as_ref[hh:hh + 1, :], axis=1, keepdims=True))
        dsts.append(jnp.sum(hs * ad_ref[hh:hh + 1, :], axis=1, keepdims=True))
    asrc_ref[...] = jnp.concatenate(srcs, axis=1)
    adst_ref[...] = jnp.concatenate(dsts, axis=1)


def _mm(x, W, a_s, a_d):
    BN = 1024
    F = x.shape[1]
    grid = (NPAD // BN,)
    return pl.pallas_call(
        _mm_body,
        grid=grid,
        in_specs=[
            pl.BlockSpec((BN, F), lambda i: (i, 0)),
            pl.BlockSpec((F, H * C), lambda i: (0, 0)),
            pl.BlockSpec((H, C), lambda i: (0, 0)),
            pl.BlockSpec((H, C), lambda i: (0, 0)),
        ],
        out_specs=[
            pl.BlockSpec((BN, H * C), lambda i: (i, 0)),
            pl.BlockSpec((BN, H), lambda i: (i, 0)),
            pl.BlockSpec((BN, H), lambda i: (i, 0)),
        ],
        out_shape=[
            jax.ShapeDtypeStruct((NPAD, H * C), jnp.float32),
            jax.ShapeDtypeStruct((NPAD, H), jnp.float32),
            jax.ShapeDtypeStruct((NPAD, H), jnp.float32),
        ],
    )(x, W, a_s, a_d)


def _mom_body(a_ref, out_ref, acc_ref):
    i = pl.program_id(0)
    nb = pl.num_programs(0)

    @pl.when(i == 0)
    def _init():
        acc_ref[...] = jnp.zeros_like(acc_ref)

    ab = a_ref[...]
    acc_ref[0:1, :] += jnp.sum(ab, axis=0, keepdims=True)
    acc_ref[1:2, :] += jnp.sum(ab * ab, axis=0, keepdims=True)

    @pl.when(i == nb - 1)
    def _fin():
        out_ref[...] = acc_ref[...]


def _moments(a):
    BN = 2048
    return pl.pallas_call(
        _mom_body,
        grid=(NPAD // BN,),
        in_specs=[pl.BlockSpec((BN, C), lambda i: (i, 0))],
        out_specs=pl.BlockSpec((8, C), lambda i: (0, 0)),
        out_shape=jax.ShapeDtypeStruct((8, C), jnp.float32),
        scratch_shapes=[pltpu.VMEM((8, C), jnp.float32)],
    )(a)


def _apply_body(a_ref, st_ref, g_ref, be_ref, o_ref):
    a = a_ref[...]
    s = st_ref[0:1, :]
    sq = st_ref[1:2, :]
    m = s * (1.0 / N)
    v = sq * (1.0 / N) - m * m
    inv = lax.rsqrt(v + EPS_BN)
    o_ref[...] = jnp.maximum((a - m) * inv * g_ref[...] + be_ref[...], 0.0)


def _bn_relu(a, stats, g, be):
    BN = 2048
    return pl.pallas_call(
        _apply_body,
        grid=(NPAD // BN,),
        in_specs=[
            pl.BlockSpec((BN, C), lambda i: (i, 0)),
            pl.BlockSpec((8, C), lambda i: (0, 0)),
            pl.BlockSpec((1, C), lambda i: (0, 0)),
            pl.BlockSpec((1, C), lambda i: (0, 0)),
        ],
        out_specs=pl.BlockSpec((BN, C), lambda i: (i, 0)),
        out_shape=jax.ShapeDtypeStruct((NPAD, C), jnp.float32),
    )(a, stats, g.reshape(1, C), be.reshape(1, C))


def _pool_fc_body(h_ref, batch_ref, fcW_ref, fcb_ref, out_ref, acc_ref):
    i = pl.program_id(0)
    nb = pl.num_programs(0)

    @pl.when(i == 0)
    def _init():
        acc_ref[...] = jnp.zeros_like(acc_ref)

    hblk = h_ref[...]
    b = batch_ref[...].reshape(1, -1)
    onehot = (b.T == lax.broadcasted_iota(jnp.int32, (1, G), 1)).astype(
        jnp.float32)
    acc_ref[:G, :] += jnp.dot(onehot.T, hblk,
                              preferred_element_type=jnp.float32)
    acc_ref[G:, :1] += jnp.sum(onehot, axis=0, keepdims=True).T

    @pl.when(i == nb - 1)
    def _fin():
        s = acc_ref[:G, :]
        cnt = acc_ref[G:, :1]
        rep = s / jnp.maximum(cnt, 1.0)
        out_ref[...] = jnp.dot(rep, fcW_ref[...],
                               preferred_element_type=jnp.float32) + fcb_ref[...]


def _pool_fc(h, batch_pad, fcW, fcb):
    BN = 2048
    grid = (NPAD // BN,)
    return pl.pallas_call(
        _pool_fc_body,
        grid=grid,
        in_specs=[
            pl.BlockSpec((BN, C), lambda i: (i, 0)),
            pl.BlockSpec((1, 1, BN), lambda i: (i, 0, 0)),
            pl.BlockSpec((C, NHID), lambda i: (0, 0)),
            pl.BlockSpec((1, NHID), lambda i: (0, 0)),
        ],
        out_specs=pl.BlockSpec((G, NHID), lambda i: (0, 0)),
        out_shape=jax.ShapeDtypeStruct((G, NHID), jnp.float32),
        scratch_shapes=[pltpu.VMEM((2 * G, C), jnp.float32)],
    )(h, batch_pad.reshape(NPAD // BN, 1, BN), fcW, fcb.reshape(1, NHID))


def kernel(x, edge_index, batch, W1, as1, ad1, b1, g1, be1, W2, as2, ad2, b2,
           g2, be2, W3, as3, ad3, b3, g3, be3, fcW, fcb):
    loop = jnp.arange(N, dtype=edge_index.dtype)
    src = jnp.concatenate([edge_index[0], loop])
    dst = jnp.concatenate([edge_index[1], loop])
    ds_sorted, ss_sorted = lax.sort((dst, src), num_keys=1)
    p = jnp.searchsorted(ds_sorted,
                         jnp.arange(NPAD + 1, dtype=jnp.int32)).astype(jnp.int32)
    ss_pad = jnp.zeros((SSLEN,), jnp.int32).at[:ET].set(ss_sorted)
    widx = jnp.clip(NPW * jnp.arange(NW, dtype=jnp.int32)[:, None]
                    + jnp.arange(RPW, dtype=jnp.int32)[None, :], 0, NPAD)
    rp = p[widx]

    xp = jnp.zeros((NPAD, F_IN), jnp.float32).at[:N].set(x)
    batch_pad = jnp.concatenate(
        [batch, jnp.full((NPAD - N,), G, dtype=batch.dtype)])

    xin = xp
    for (W, a_s, a_d, g, be) in ((W1, as1, ad1, g1, be1),
                                 (W2, as2, ad2, g2, be2),
                                 (W3, as3, ad3, g3, be3)):
        h, asrc, adst = _mm(xin, W, a_s, a_d)
        agg = _sc_layer(h, asrc.reshape(-1), adst.reshape(-1), ss_pad, rp)
        stats = _moments(agg)
        xin = _bn_relu(agg, stats, g, be)

    return _pool_fc(xin, batch_pad, fcW, fcb)
